# baseline (speedup 1.0000x reference)
import functools

import jax
import jax.numpy as jnp
from jax.experimental import pallas as pl
from jax.experimental.pallas import tpu as pltpu

_N = 10000
_BLK = 1000
_NB = _N // _BLK
_F = 128
_MININT = -(2 ** 31)
_KS = (5000, 2500, 1250)


def _f32_key(s):
    i = jax.lax.bitcast_convert_type(s, jnp.int32)
    return jnp.where(i >= 0, i, i ^ jnp.int32(0x7FFFFFFF))


def _kth_largest_thresh(keys, k):
    def body(i, t_u):
        bit = jax.lax.shift_left(jnp.int32(1), jnp.int32(31) - i)
        t2 = t_u | bit
        thresh = t2 ^ jnp.int32(_MININT)
        cnt = jnp.sum((keys >= thresh).astype(jnp.int32))
        return jnp.where(cnt >= k, t2, t_u)

    t_u = jax.lax.fori_loop(0, 32, body, jnp.int32(0))
    return t_u ^ jnp.int32(_MININT)


def _fused_kernel(x_ref, w1_ref, b1_ref, w2_ref, b2_ref, p_ref, out_ref,
                  x0_s, q_s):
    i = pl.program_id(0)
    rows = pl.ds(i * _BLK, _BLK)
    h = jnp.maximum(
        jnp.dot(x_ref[...], w1_ref[...], preferred_element_type=jnp.float32)
        + b1_ref[...], 0.0)
    x0 = jnp.dot(h, w2_ref[...], preferred_element_type=jnp.float32) + b2_ref[...]
    x0_s[rows, :] = x0
    q_s[rows, :] = jnp.dot(x0, p_ref[...], preferred_element_type=jnp.float32)

    @pl.when(i == _NB - 1)
    def _select_and_reduce():
        q = q_s[...]
        col = jax.lax.broadcasted_iota(jnp.int32, (_N, 8), 1)
        w = jnp.ones((_N, 1), jnp.float32)
        alive = jnp.ones((_N, 1), jnp.bool_)
        for r, k in enumerate(_KS):
            s = w * q[:, r:r + 1]
            mask = (col == r) & alive
            keys = jnp.where(mask, _f32_key(jnp.broadcast_to(s, (_N, 8))),
                             jnp.int32(_MININT))
            t = _kth_largest_thresh(keys, k)
            picked = keys >= t
            alive = jnp.sum(picked.astype(jnp.int32), axis=1, keepdims=True) > 0
            w = jnp.where(alive, w * jnp.tanh(s), 0.0)
        out_ref[...] = jax.lax.dot_general(
            w, x0_s[...], (((0,), (0,)), ((), ())),
            preferred_element_type=jnp.float32) * (1.0 / _KS[-1])


def kernel(x, edge_index, edge_attr, batch, params):
    del edge_index, edge_attr, batch
    nm = params['node_mlp']
    pool = params['pool']
    p = jnp.stack([pv / jnp.linalg.norm(pv) for pv in pool], axis=1)
    p = jnp.pad(p, ((0, 0), (0, 8 - len(pool))))
    full = lambda shape: pl.BlockSpec(shape, lambda i: (0, 0))
    out = pl.pallas_call(
        _fused_kernel,
        grid=(_NB,),
        in_specs=[
            pl.BlockSpec((_BLK, _F), lambda i: (i, 0)),
            full((_F, _F)), full((1, _F)), full((_F, _F)), full((1, _F)),
            full((_F, 8)),
        ],
        out_specs=full((1, _F)),
        out_shape=jax.ShapeDtypeStruct((1, _F), jnp.float32),
        scratch_shapes=[
            pltpu.VMEM((_N, _F), jnp.float32),
            pltpu.VMEM((_N, 8), jnp.float32),
        ],
    )(x, nm['W1'], nm['b1'][None, :], nm['W2'], nm['b2'][None, :], p)
    return out


# speedup vs baseline: 4.8335x; 4.8335x over previous
import jax
import jax.numpy as jnp
from jax.experimental import pallas as pl
from jax.experimental.pallas import tpu as pltpu

_N = 10000
_NP = 10240
_BLK = 1024
_NB = _NP // _BLK
_F = 128
_MININT = -(2 ** 31)
_KS = (5000, 2500, 1250)


def _f32_key(s):
    i = jax.lax.bitcast_convert_type(s, jnp.int32)
    return jnp.where(i >= 0, i, i ^ jnp.int32(0x7FFFFFFF))


def _kth_largest_thresh(keys8, k):
    d = jax.lax.broadcasted_iota(jnp.int32, (8, 1), 0)
    t_u = jnp.int32(0)
    for shift in (29, 26, 23, 20, 17, 14, 11, 8, 5, 2):
        cand = t_u | (d << shift)
        thr = cand ^ jnp.int32(_MININT)
        cnt = jnp.sum((keys8 >= thr).astype(jnp.int32), axis=1, keepdims=True)
        dstar = jnp.sum(((cnt >= k) & (d >= 1)).astype(jnp.int32))
        t_u = t_u | (dstar << shift)
    cand = t_u | d
    thr = cand ^ jnp.int32(_MININT)
    cnt = jnp.sum((keys8 >= thr).astype(jnp.int32), axis=1, keepdims=True)
    dstar = jnp.sum(((cnt >= k) & (d >= 1) & (d <= 3)).astype(jnp.int32))
    return (t_u | dstar) ^ jnp.int32(_MININT)


def _fused_kernel(x_ref, w1_ref, b1_ref, w2_ref, b2_ref, p_ref, out_ref,
                  x0_s, qt_s):
    i = pl.program_id(0)
    h = jnp.maximum(
        jnp.dot(x_ref[...], w1_ref[...], preferred_element_type=jnp.float32)
        + b1_ref[...], 0.0)
    x0 = jnp.dot(h, w2_ref[...], preferred_element_type=jnp.float32) + b2_ref[...]
    x0_s[pl.ds(i * _BLK, _BLK), :] = x0
    qt_s[:, pl.ds(i * _BLK, _BLK)] = jax.lax.dot_general(
        p_ref[...], x0, (((1,), (1,)), ((), ())),
        preferred_element_type=jnp.float32)

    @pl.when(i == _NB - 1)
    def _select_and_reduce():
        w = jnp.ones((1, _NP), jnp.float32)
        alive = jax.lax.broadcasted_iota(jnp.int32, (1, _NP), 1) < _N
        for r, k in enumerate(_KS):
            s = w * qt_s[r:r + 1, :]
            keys = jnp.where(alive, _f32_key(s), jnp.int32(_MININT))
            t = _kth_largest_thresh(jnp.broadcast_to(keys, (8, _NP)), k)
            alive = keys >= t
            w = jnp.where(alive, w * jnp.tanh(s), 0.0)
        out_ref[...] = jax.lax.dot_general(
            w, x0_s[...], (((1,), (0,)), ((), ())),
            preferred_element_type=jnp.float32) * (1.0 / _KS[-1])


def kernel(x, edge_index, edge_attr, batch, params):
    del edge_index, edge_attr, batch
    nm = params['node_mlp']
    pool = params['pool']
    p = jnp.stack([pv / jnp.linalg.norm(pv) for pv in pool], axis=0)
    p = jnp.pad(p, ((0, 8 - len(pool)), (0, 0)))
    xp = jnp.pad(x, ((0, _NP - _N), (0, 0)))
    full = lambda shape: pl.BlockSpec(shape, lambda i: (0, 0))
    out = pl.pallas_call(
        _fused_kernel,
        grid=(_NB,),
        in_specs=[
            pl.BlockSpec((_BLK, _F), lambda i: (i, 0)),
            full((_F, _F)), full((1, _F)), full((_F, _F)), full((1, _F)),
            full((8, _F)),
        ],
        out_specs=full((1, _F)),
        out_shape=jax.ShapeDtypeStruct((1, _F), jnp.float32),
        scratch_shapes=[
            pltpu.VMEM((_NP, _F), jnp.float32),
            pltpu.VMEM((8, _NP), jnp.float32),
        ],
    )(xp, nm['W1'], nm['b1'][None, :], nm['W2'], nm['b2'][None, :], p)
    return out
